# baseline (speedup 1.0000x reference)
import functools

import jax
import jax.numpy as jnp
from jax import lax
from jax.experimental import pallas as pl
from jax.experimental.pallas import tpu as pltpu
from jax.experimental.pallas import tpu_sc as plsc


_NC = 2
_NS = 16
_NW = _NC * _NS
_N_ROWS = 320000
_RPW = _N_ROWS // _NW
_CHUNK = 200
_NBUF = 2
_NCHUNK = _RPW // _CHUNK


def _sc_colsum_body(x_hbm, out_hbm, buf_ref, acc_ref, sem0, sem1):
    c = lax.axis_index("c")
    s = lax.axis_index("s")
    wid = s * _NC + c
    base = wid * _RPW
    sems = (sem0, sem1)

    for b in range(_NBUF):
        pltpu.make_async_copy(
            x_hbm.at[pl.ds(base + b * _CHUNK, _CHUNK)], buf_ref.at[b], sems[b]
        ).start()

    def outer(g, acc):
        for b in range(_NBUF):
            chunk = g * _NBUF + b
            row0 = base + chunk * _CHUNK
            pltpu.make_async_copy(
                x_hbm.at[pl.ds(row0, _CHUNK)], buf_ref.at[b], sems[b]
            ).wait()

            def inner(r, a):
                return tuple(
                    a[j] + buf_ref[b, r, pl.ds(16 * j, 16)] for j in range(8)
                )

            acc = lax.fori_loop(0, _CHUNK, inner, acc)

            nxt = chunk + _NBUF

            @pl.when(nxt < _NCHUNK)
            def _prefetch():
                pltpu.make_async_copy(
                    x_hbm.at[pl.ds(base + nxt * _CHUNK, _CHUNK)],
                    buf_ref.at[b],
                    sems[b],
                ).start()

        return acc

    acc0 = tuple(jnp.zeros((16,), jnp.float32) for _ in range(8))
    acc = lax.fori_loop(0, _NCHUNK // _NBUF, outer, acc0)

    for j in range(8):
        acc_ref[pl.ds(16 * j, 16)] = acc[j]
    pltpu.sync_copy(acc_ref, out_hbm.at[wid])


_sc_colsum = pl.kernel(
    _sc_colsum_body,
    out_type=jax.ShapeDtypeStruct((_NW, 128), jnp.float32),
    mesh=plsc.VectorSubcoreMesh(core_axis_name="c", subcore_axis_name="s"),
    scratch_types=[
        pltpu.VMEM((_NBUF, _CHUNK, 128), jnp.float32),
        pltpu.VMEM((128,), jnp.float32),
        pltpu.SemaphoreType.DMA,
        pltpu.SemaphoreType.DMA,
    ],
)


def kernel(x):
    partials = _sc_colsum(x)
    return partials.sum(axis=0, keepdims=True)
